# baseline (speedup 1.0000x reference)
import functools

import jax
import jax.numpy as jnp
from jax import lax
from jax.experimental import pallas as pl
from jax.experimental.pallas import tpu as pltpu
from jax.experimental.pallas import tpu_sc as plsc

N = 10000
E = 320000
D = 128
H = 128
G = 16
GF = 1
OUT = 1

NC = 2
NS = 16
NW = NC * NS
EPW = E // NW
CH = 80
NCH = EPW // CH
CHS = 80
PADE = 80
EPWP = EPW + PADE
NCHS = EPWP // CHS
NPR = NCHS // 2
RPT = 624
NTAIL = N - NS * RPT

_F32 = jnp.float32
_HIGH = lax.Precision.HIGHEST

@functools.cache
def _mesh():
    return plsc.VectorSubcoreMesh(
        core_axis_name="c", subcore_axis_name="s", num_cores=NC, num_subcores=NS
    )


def _sc_degree(dst, ones128, z128):
    @functools.partial(
        pl.kernel,
        out_type=jax.ShapeDtypeStruct((NC, N, D), _F32),
        mesh=_mesh(),
        scratch_types=[
            pltpu.VMEM((NCH, CH), jnp.int32),
            pltpu.VMEM((CH, D), _F32),
            pltpu.VMEM_SHARED((N, D), _F32),
            pltpu.SemaphoreType.DMA,
        ],
    )
    def deg_kernel(dst_hbm, ones_hbm, z16_hbm, out_hbm, dsts, ones_v, hist, sem):
        c = lax.axis_index("c")
        s = lax.axis_index("s")
        wid = c * NS + s
        pltpu.sync_copy(z16_hbm.at[pl.ds(s * RPT, RPT)], hist.at[pl.ds(s * RPT, RPT)])

        @pl.when(s == NS - 1)
        def _():
            pltpu.sync_copy(
                z16_hbm.at[pl.ds(NS * RPT, NTAIL)], hist.at[pl.ds(NS * RPT, NTAIL)]
            )

        pltpu.sync_copy(ones_hbm, ones_v)
        pltpu.sync_copy(dst_hbm.at[wid], dsts)
        plsc.subcore_barrier()

        @pl.loop(0, NCH)
        def _(k):
            pltpu.async_copy(ones_v, hist.at[dsts.at[k]], sem, add=True)

        @pl.loop(0, NCH)
        def _(k):
            pltpu.make_async_copy(ones_v, hist.at[dsts.at[k]], sem).wait()

        plsc.subcore_barrier()
        pltpu.sync_copy(
            hist.at[pl.ds(s * RPT, RPT)],
            out_hbm.at[c, pl.ds(s * RPT, RPT)],
        )

        @pl.when(s == NS - 1)
        def _():
            pltpu.sync_copy(
                hist.at[pl.ds(NS * RPT, NTAIL)],
                out_hbm.at[c, pl.ds(NS * RPT, NTAIL)],
            )

    return deg_kernel(dst, ones128, z128)


def _sc_scatter(hs, eidx, z128):
    @functools.partial(
        pl.kernel,
        out_type=jax.ShapeDtypeStruct((NC, N, D), _F32),
        mesh=_mesh(),
        scratch_types=[
            pltpu.VMEM((2, 2, 2, CHS), jnp.int32),
            pltpu.VMEM((2, CHS, D), _F32),
            pltpu.VMEM_SHARED((N + 8, D), _F32),
            pltpu.SemaphoreType.DMA,
            pltpu.SemaphoreType.DMA,
            pltpu.SemaphoreType.DMA,
            pltpu.SemaphoreType.DMA,
        ],
    )
    def scatter_kernel(
        hs_hbm, eidx_hbm, z_hbm, out_hbm, ei, rows, acc, sem0, sem1, semi0, semi1
    ):
        c = lax.axis_index("c")
        s = lax.axis_index("s")
        wid = c * NS + s
        pltpu.sync_copy(z_hbm.at[pl.ds(s * RPT, RPT)], acc.at[pl.ds(s * RPT, RPT)])

        @pl.when(s == NS - 1)
        def _():
            pltpu.sync_copy(
                z_hbm.at[pl.ds(NS * RPT, NTAIL)], acc.at[pl.ds(NS * RPT, NTAIL)]
            )

        plsc.subcore_barrier()

        pltpu.sync_copy(eidx_hbm.at[wid, pl.ds(0, 2)], ei.at[0])
        pltpu.async_copy(eidx_hbm.at[wid, pl.ds(2, 2)], ei.at[1], semi1)
        pltpu.async_copy(hs_hbm.at[ei.at[0, 0, 0]], rows.at[0], sem0)

        @pl.loop(0, NPR - 1, step=2)
        def _(p):
            pltpu.make_async_copy(hs_hbm.at[ei.at[0, 0, 0]], rows.at[0], sem0).wait()
            pltpu.async_copy(hs_hbm.at[ei.at[0, 1, 0]], rows.at[1], sem1)
            pltpu.sync_copy(rows.at[0], acc.at[ei.at[0, 0, 1]], add=True)
            pltpu.make_async_copy(
                eidx_hbm.at[wid, pl.ds(2 * (p + 1), 2)], ei.at[1], semi1
            ).wait()
            pltpu.make_async_copy(hs_hbm.at[ei.at[0, 1, 0]], rows.at[1], sem1).wait()
            pltpu.async_copy(hs_hbm.at[ei.at[1, 0, 0]], rows.at[0], sem0)
            pltpu.sync_copy(rows.at[1], acc.at[ei.at[0, 1, 1]], add=True)
            pltpu.async_copy(eidx_hbm.at[wid, pl.ds(2 * (p + 2), 2)], ei.at[0], semi0)
            pltpu.make_async_copy(hs_hbm.at[ei.at[1, 0, 0]], rows.at[0], sem0).wait()
            pltpu.async_copy(hs_hbm.at[ei.at[1, 1, 0]], rows.at[1], sem1)
            pltpu.sync_copy(rows.at[0], acc.at[ei.at[1, 0, 1]], add=True)
            pltpu.make_async_copy(
                eidx_hbm.at[wid, pl.ds(2 * (p + 2), 2)], ei.at[0], semi0
            ).wait()
            pltpu.make_async_copy(hs_hbm.at[ei.at[1, 1, 0]], rows.at[1], sem1).wait()
            pltpu.async_copy(hs_hbm.at[ei.at[0, 0, 0]], rows.at[0], sem0)
            pltpu.sync_copy(rows.at[1], acc.at[ei.at[1, 1, 1]], add=True)

            @pl.when(p + 3 < NPR)
            def _():
                pltpu.async_copy(
                    eidx_hbm.at[wid, pl.ds(2 * (p + 3), 2)], ei.at[1], semi1
                )

        pltpu.make_async_copy(hs_hbm.at[ei.at[0, 0, 0]], rows.at[0], sem0).wait()
        pltpu.async_copy(hs_hbm.at[ei.at[0, 1, 0]], rows.at[1], sem1)
        pltpu.sync_copy(rows.at[0], acc.at[ei.at[0, 0, 1]], add=True)
        pltpu.make_async_copy(hs_hbm.at[ei.at[0, 1, 0]], rows.at[1], sem1).wait()
        pltpu.sync_copy(rows.at[1], acc.at[ei.at[0, 1, 1]], add=True)

        plsc.subcore_barrier()
        pltpu.sync_copy(
            acc.at[pl.ds(s * RPT, RPT)],
            out_hbm.at[c, pl.ds(s * RPT, RPT)],
        )

        @pl.when(s == NS - 1)
        def _():
            pltpu.sync_copy(
                acc.at[pl.ds(NS * RPT, NTAIL)],
                out_hbm.at[c, pl.ds(NS * RPT, NTAIL)],
            )

    return scatter_kernel(hs, eidx, z128)


RB = 1000
NB = N // RB

_row = pl.BlockSpec((RB, H), lambda i: (i, 0))
_degs = pl.BlockSpec((NC, RB, D), lambda i: (0, i, 0))
_part = pl.BlockSpec((NC, RB, D), lambda i: (0, i, 0))
_wspec = pl.BlockSpec((H, H), lambda i: (0, 0))
_bspec = pl.BlockSpec((1, H), lambda i: (0, 0))


def _deg_cols(degp_ref):
    deg = degp_ref[0, :, 0:1] + degp_ref[1, :, 0:1] + 1.0
    dinv = lax.rsqrt(deg)
    ideg = 1.0 / deg
    return dinv, ideg


def _mm_body(x_ref, w_ref, o_ref):
    o_ref[...] = jnp.dot(
        x_ref[...], w_ref[...], precision=_HIGH, preferred_element_type=_F32
    )


def _tc_mm(x, W1):
    return pl.pallas_call(
        _mm_body,
        grid=(NB,),
        in_specs=[pl.BlockSpec((RB, D), lambda i: (i, 0)), _wspec],
        out_specs=_row,
        out_shape=jax.ShapeDtypeStruct((N, H), _F32),
    )(x, W1)


def _prep_body(h_ref, degp_ref, b_ref, hs_ref, self_ref, dinv_ref):
    dinv, ideg = _deg_cols(degp_ref)
    h = h_ref[...]
    hs_ref[...] = h * dinv
    self_ref[...] = h * ideg + b_ref[...]
    dinv_ref[...] = jnp.broadcast_to(dinv, (RB, 8))


def _tc_prep(h1, degp, b1):
    return pl.pallas_call(
        _prep_body,
        grid=(NB,),
        in_specs=[_row, _degs, _bspec],
        out_specs=(_row, _row, pl.BlockSpec((RB, 8), lambda i: (i, 0))),
        out_shape=(
            jax.ShapeDtypeStruct((N, H), _F32),
            jax.ShapeDtypeStruct((N, H), _F32),
            jax.ShapeDtypeStruct((N, 8), _F32),
        ),
    )(h1, degp, b1)


def _combine_body(ap_ref, dinv_ref, self1_ref, w2_ref, b2_ref, hs2_ref, self2_ref):
    dinv = dinv_ref[:, 0:1]
    ideg = dinv * dinv
    a = ap_ref[0] + ap_ref[1]
    h2in = jnp.maximum(dinv * a + self1_ref[...], 0.0)
    h2 = jnp.dot(h2in, w2_ref[...], precision=_HIGH, preferred_element_type=_F32)
    hs2_ref[...] = h2 * dinv
    self2_ref[...] = h2 * ideg + b2_ref[...]


def _tc_combine(ap, dinv8, self1, W2, b2):
    return pl.pallas_call(
        _combine_body,
        grid=(NB,),
        in_specs=[_part, pl.BlockSpec((RB, 8), lambda i: (i, 0)), _row, _wspec, _bspec],
        out_specs=(_row, _row),
        out_shape=(
            jax.ShapeDtypeStruct((N, H), _F32),
            jax.ShapeDtypeStruct((N, H), _F32),
        ),
    )(ap, dinv8, self1, W2, b2)


def _final_body(bp_ref, dinv_ref, self2_ref, batch_ref, gf_ref, wl_ref, bl_ref, o_ref):
    i = pl.program_id(0)
    dinv = dinv_ref[:, 0:1]
    b = bp_ref[0] + bp_ref[1]
    out2 = jnp.maximum(dinv * b + self2_ref[...], 0.0)
    seg = batch_ref[0]
    gids = lax.broadcasted_iota(jnp.int32, (G, RB), 0)
    M = (seg == gids).astype(_F32)
    pooled = jnp.dot(M, out2, precision=_HIGH, preferred_element_type=_F32)
    wl = wl_ref[...]
    part = jnp.dot(pooled, wl[0:H, :], precision=_HIGH, preferred_element_type=_F32)

    @pl.when(i == 0)
    def _():
        o_ref[...] = gf_ref[...] * wl[H : H + 1, :] + bl_ref[...]

    o_ref[...] += part


def _tc_final(bp, dinv8, self2, batch2d, graph_feat, Wlin, blin2d):
    return pl.pallas_call(
        _final_body,
        grid=(NB,),
        in_specs=[
            _part,
            pl.BlockSpec((RB, 8), lambda i: (i, 0)),
            _row,
            pl.BlockSpec((1, 1, RB), lambda i: (i, 0, 0)),
            pl.BlockSpec((G, GF), lambda i: (0, 0)),
            pl.BlockSpec((H + GF, OUT), lambda i: (0, 0)),
            pl.BlockSpec((1, OUT), lambda i: (0, 0)),
        ],
        out_specs=pl.BlockSpec((G, OUT), lambda i: (0, 0)),
        out_shape=jax.ShapeDtypeStruct((G, OUT), _F32),
    )(bp, dinv8, self2, batch2d, graph_feat, Wlin, blin2d)


def kernel(x, edge_index, batch, graph_feat, W1, b1, W2, b2, Wlin, blin):
    dst = edge_index[1].reshape(NW, NCH, CH)
    pad_s = jnp.zeros((NW, PADE, 1, 1), jnp.int32)
    pad_d = jnp.full((NW, PADE, 1, 1), N, jnp.int32)
    ew = edge_index.reshape(2, NW, EPW).transpose(1, 2, 0)
    pad = jnp.concatenate([pad_s, pad_d], 3).reshape(NW, PADE, 2)
    eidx = jnp.concatenate([ew, pad], 1).reshape(NW, NCHS, CHS, 2)
    eidx = eidx.transpose(0, 1, 3, 2)
    z128 = jnp.zeros((N, D), _F32)
    ones128 = jnp.ones((CH, D), _F32)

    degp = _sc_degree(dst, ones128, z128)
    h1 = _tc_mm(x, W1)
    hs1, self1, dinv8 = _tc_prep(h1, degp, b1.reshape(1, H))
    ap = _sc_scatter(hs1, eidx, z128)
    hs2, self2 = _tc_combine(ap, dinv8, self1, W2, b2.reshape(1, H))
    bp = _sc_scatter(hs2, eidx, z128)
    return _tc_final(
        bp, dinv8, self2, batch.reshape(NB, 1, RB), graph_feat, Wlin,
        blin.reshape(1, OUT),
    )


# speedup vs baseline: 1.0447x; 1.0447x over previous
import functools

import jax
import jax.numpy as jnp
from jax import lax
from jax.experimental import pallas as pl
from jax.experimental.pallas import tpu as pltpu
from jax.experimental.pallas import tpu_sc as plsc

N = 10000
E = 320000
D = 128
H = 128
G = 16
GF = 1
OUT = 1

NC = 2
NS = 16
NW = NC * NS
EPW = E // NW
CH = 80
NCH = EPW // CH
CHS = 40
PADE = 0
EPWP = EPW + PADE
NCHS = EPWP // CHS
NPR = NCHS // 2
RPT = 624
NTAIL = N - NS * RPT

_F32 = jnp.float32
_HIGH = lax.Precision.HIGHEST

@functools.cache
def _mesh():
    return plsc.VectorSubcoreMesh(
        core_axis_name="c", subcore_axis_name="s", num_cores=NC, num_subcores=NS
    )


def _sc_degree(dst, ones128, z128):
    @functools.partial(
        pl.kernel,
        out_type=jax.ShapeDtypeStruct((NC, N, D), _F32),
        mesh=_mesh(),
        scratch_types=[
            pltpu.VMEM((NCH, CH), jnp.int32),
            pltpu.VMEM((CH, D), _F32),
            pltpu.VMEM_SHARED((N, D), _F32),
            pltpu.SemaphoreType.DMA,
        ],
    )
    def deg_kernel(dst_hbm, ones_hbm, z16_hbm, out_hbm, dsts, ones_v, hist, sem):
        c = lax.axis_index("c")
        s = lax.axis_index("s")
        wid = c * NS + s
        pltpu.sync_copy(z16_hbm.at[pl.ds(s * RPT, RPT)], hist.at[pl.ds(s * RPT, RPT)])

        @pl.when(s == NS - 1)
        def _():
            pltpu.sync_copy(
                z16_hbm.at[pl.ds(NS * RPT, NTAIL)], hist.at[pl.ds(NS * RPT, NTAIL)]
            )

        pltpu.sync_copy(ones_hbm, ones_v)
        pltpu.sync_copy(dst_hbm.at[wid], dsts)
        plsc.subcore_barrier()

        @pl.loop(0, NCH)
        def _(k):
            pltpu.async_copy(ones_v, hist.at[dsts.at[k]], sem, add=True)

        @pl.loop(0, NCH)
        def _(k):
            pltpu.make_async_copy(ones_v, hist.at[dsts.at[k]], sem).wait()

        plsc.subcore_barrier()
        pltpu.sync_copy(
            hist.at[pl.ds(s * RPT, RPT)],
            out_hbm.at[c, pl.ds(s * RPT, RPT)],
        )

        @pl.when(s == NS - 1)
        def _():
            pltpu.sync_copy(
                hist.at[pl.ds(NS * RPT, NTAIL)],
                out_hbm.at[c, pl.ds(NS * RPT, NTAIL)],
            )

    return deg_kernel(dst, ones128, z128)


def _sc_scatter(hs, eidx, z128):
    @functools.partial(
        pl.kernel,
        out_type=jax.ShapeDtypeStruct((NC, N, D), _F32),
        mesh=_mesh(),
        scratch_types=[
            pltpu.VMEM((2, 2, 2, CHS), jnp.int32),
            pltpu.VMEM((2, CHS, D), _F32),
            pltpu.VMEM_SHARED((N + 8, D), _F32),
            pltpu.SemaphoreType.DMA,
            pltpu.SemaphoreType.DMA,
            pltpu.SemaphoreType.DMA,
            pltpu.SemaphoreType.DMA,
        ],
    )
    def scatter_kernel(
        hs_hbm, eidx_hbm, z_hbm, out_hbm, ei, rows, acc, sem0, sem1, semi0, semi1
    ):
        c = lax.axis_index("c")
        s = lax.axis_index("s")
        wid = c * NS + s
        pltpu.sync_copy(z_hbm.at[pl.ds(s * RPT, RPT)], acc.at[pl.ds(s * RPT, RPT)])

        @pl.when(s == NS - 1)
        def _():
            pltpu.sync_copy(
                z_hbm.at[pl.ds(NS * RPT, NTAIL)], acc.at[pl.ds(NS * RPT, NTAIL)]
            )

        plsc.subcore_barrier()

        pltpu.sync_copy(eidx_hbm.at[wid, pl.ds(0, 2)], ei.at[0])
        pltpu.async_copy(eidx_hbm.at[wid, pl.ds(2, 2)], ei.at[1], semi1)
        pltpu.async_copy(hs_hbm.at[ei.at[0, 0, 0]], rows.at[0], sem0)

        @pl.loop(0, NPR - 1, step=2)
        def _(p):
            pltpu.make_async_copy(hs_hbm.at[ei.at[0, 0, 0]], rows.at[0], sem0).wait()
            pltpu.async_copy(hs_hbm.at[ei.at[0, 1, 0]], rows.at[1], sem1)
            pltpu.sync_copy(rows.at[0], acc.at[ei.at[0, 0, 1]], add=True)
            pltpu.make_async_copy(
                eidx_hbm.at[wid, pl.ds(2 * (p + 1), 2)], ei.at[1], semi1
            ).wait()
            pltpu.make_async_copy(hs_hbm.at[ei.at[0, 1, 0]], rows.at[1], sem1).wait()
            pltpu.async_copy(hs_hbm.at[ei.at[1, 0, 0]], rows.at[0], sem0)
            pltpu.sync_copy(rows.at[1], acc.at[ei.at[0, 1, 1]], add=True)
            pltpu.async_copy(eidx_hbm.at[wid, pl.ds(2 * (p + 2), 2)], ei.at[0], semi0)
            pltpu.make_async_copy(hs_hbm.at[ei.at[1, 0, 0]], rows.at[0], sem0).wait()
            pltpu.async_copy(hs_hbm.at[ei.at[1, 1, 0]], rows.at[1], sem1)
            pltpu.sync_copy(rows.at[0], acc.at[ei.at[1, 0, 1]], add=True)
            pltpu.make_async_copy(
                eidx_hbm.at[wid, pl.ds(2 * (p + 2), 2)], ei.at[0], semi0
            ).wait()
            pltpu.make_async_copy(hs_hbm.at[ei.at[1, 1, 0]], rows.at[1], sem1).wait()
            pltpu.async_copy(hs_hbm.at[ei.at[0, 0, 0]], rows.at[0], sem0)
            pltpu.sync_copy(rows.at[1], acc.at[ei.at[1, 1, 1]], add=True)

            @pl.when(p + 3 < NPR)
            def _():
                pltpu.async_copy(
                    eidx_hbm.at[wid, pl.ds(2 * (p + 3), 2)], ei.at[1], semi1
                )

        pltpu.make_async_copy(hs_hbm.at[ei.at[0, 0, 0]], rows.at[0], sem0).wait()
        pltpu.async_copy(hs_hbm.at[ei.at[0, 1, 0]], rows.at[1], sem1)
        pltpu.sync_copy(rows.at[0], acc.at[ei.at[0, 0, 1]], add=True)
        pltpu.make_async_copy(hs_hbm.at[ei.at[0, 1, 0]], rows.at[1], sem1).wait()
        pltpu.sync_copy(rows.at[1], acc.at[ei.at[0, 1, 1]], add=True)

        plsc.subcore_barrier()
        pltpu.sync_copy(
            acc.at[pl.ds(s * RPT, RPT)],
            out_hbm.at[c, pl.ds(s * RPT, RPT)],
        )

        @pl.when(s == NS - 1)
        def _():
            pltpu.sync_copy(
                acc.at[pl.ds(NS * RPT, NTAIL)],
                out_hbm.at[c, pl.ds(NS * RPT, NTAIL)],
            )

    return scatter_kernel(hs, eidx, z128)


RB = 1000
NB = N // RB

_row = pl.BlockSpec((RB, H), lambda i: (i, 0))
_degs = pl.BlockSpec((NC, RB, D), lambda i: (0, i, 0))
_part = pl.BlockSpec((NC, RB, D), lambda i: (0, i, 0))
_wspec = pl.BlockSpec((H, H), lambda i: (0, 0))
_bspec = pl.BlockSpec((1, H), lambda i: (0, 0))


def _deg_cols(degp_ref):
    deg = degp_ref[0, :, 0:1] + degp_ref[1, :, 0:1] + 1.0
    dinv = lax.rsqrt(deg)
    ideg = 1.0 / deg
    return dinv, ideg


def _mm_body(x_ref, w_ref, o_ref):
    o_ref[...] = jnp.dot(
        x_ref[...], w_ref[...], precision=_HIGH, preferred_element_type=_F32
    )


def _tc_mm(x, W1):
    return pl.pallas_call(
        _mm_body,
        grid=(NB,),
        in_specs=[pl.BlockSpec((RB, D), lambda i: (i, 0)), _wspec],
        out_specs=_row,
        out_shape=jax.ShapeDtypeStruct((N, H), _F32),
    )(x, W1)


def _prep_body(h_ref, degp_ref, b_ref, hs_ref, self_ref, dinv_ref):
    dinv, ideg = _deg_cols(degp_ref)
    h = h_ref[...]
    hs_ref[...] = h * dinv
    self_ref[...] = h * ideg + b_ref[...]
    dinv_ref[...] = jnp.broadcast_to(dinv, (RB, 8))


def _tc_prep(h1, degp, b1):
    return pl.pallas_call(
        _prep_body,
        grid=(NB,),
        in_specs=[_row, _degs, _bspec],
        out_specs=(_row, _row, pl.BlockSpec((RB, 8), lambda i: (i, 0))),
        out_shape=(
            jax.ShapeDtypeStruct((N, H), _F32),
            jax.ShapeDtypeStruct((N, H), _F32),
            jax.ShapeDtypeStruct((N, 8), _F32),
        ),
    )(h1, degp, b1)


def _combine_body(ap_ref, dinv_ref, self1_ref, w2_ref, b2_ref, hs2_ref, self2_ref):
    dinv = dinv_ref[:, 0:1]
    ideg = dinv * dinv
    a = ap_ref[0] + ap_ref[1]
    h2in = jnp.maximum(dinv * a + self1_ref[...], 0.0)
    h2 = jnp.dot(h2in, w2_ref[...], precision=_HIGH, preferred_element_type=_F32)
    hs2_ref[...] = h2 * dinv
    self2_ref[...] = h2 * ideg + b2_ref[...]


def _tc_combine(ap, dinv8, self1, W2, b2):
    return pl.pallas_call(
        _combine_body,
        grid=(NB,),
        in_specs=[_part, pl.BlockSpec((RB, 8), lambda i: (i, 0)), _row, _wspec, _bspec],
        out_specs=(_row, _row),
        out_shape=(
            jax.ShapeDtypeStruct((N, H), _F32),
            jax.ShapeDtypeStruct((N, H), _F32),
        ),
    )(ap, dinv8, self1, W2, b2)


def _final_body(bp_ref, dinv_ref, self2_ref, batch_ref, gf_ref, wl_ref, bl_ref, o_ref):
    i = pl.program_id(0)
    dinv = dinv_ref[:, 0:1]
    b = bp_ref[0] + bp_ref[1]
    out2 = jnp.maximum(dinv * b + self2_ref[...], 0.0)
    seg = batch_ref[0]
    gids = lax.broadcasted_iota(jnp.int32, (G, RB), 0)
    M = (seg == gids).astype(_F32)
    pooled = jnp.dot(M, out2, precision=_HIGH, preferred_element_type=_F32)
    wl = wl_ref[...]
    part = jnp.dot(pooled, wl[0:H, :], precision=_HIGH, preferred_element_type=_F32)

    @pl.when(i == 0)
    def _():
        o_ref[...] = gf_ref[...] * wl[H : H + 1, :] + bl_ref[...]

    o_ref[...] += part


def _tc_final(bp, dinv8, self2, batch2d, graph_feat, Wlin, blin2d):
    return pl.pallas_call(
        _final_body,
        grid=(NB,),
        in_specs=[
            _part,
            pl.BlockSpec((RB, 8), lambda i: (i, 0)),
            _row,
            pl.BlockSpec((1, 1, RB), lambda i: (i, 0, 0)),
            pl.BlockSpec((G, GF), lambda i: (0, 0)),
            pl.BlockSpec((H + GF, OUT), lambda i: (0, 0)),
            pl.BlockSpec((1, OUT), lambda i: (0, 0)),
        ],
        out_specs=pl.BlockSpec((G, OUT), lambda i: (0, 0)),
        out_shape=jax.ShapeDtypeStruct((G, OUT), _F32),
    )(bp, dinv8, self2, batch2d, graph_feat, Wlin, blin2d)


def kernel(x, edge_index, batch, graph_feat, W1, b1, W2, b2, Wlin, blin):
    dst = edge_index[1].reshape(NW, NCH, CH)
    ew = edge_index.reshape(2, NW, EPW).transpose(1, 2, 0)
    if PADE:
        pad_s = jnp.zeros((NW, PADE, 1, 1), jnp.int32)
        pad_d = jnp.full((NW, PADE, 1, 1), N, jnp.int32)
        pad = jnp.concatenate([pad_s, pad_d], 3).reshape(NW, PADE, 2)
        ew = jnp.concatenate([ew, pad], 1)
    eidx = ew.reshape(NW, NCHS, CHS, 2).transpose(0, 1, 3, 2)
    z128 = jnp.zeros((N, D), _F32)
    ones128 = jnp.ones((CH, D), _F32)

    degp = _sc_degree(dst, ones128, z128)
    h1 = _tc_mm(x, W1)
    hs1, self1, dinv8 = _tc_prep(h1, degp, b1.reshape(1, H))
    ap = _sc_scatter(hs1, eidx, z128)
    hs2, self2 = _tc_combine(ap, dinv8, self1, W2, b2.reshape(1, H))
    bp = _sc_scatter(hs2, eidx, z128)
    return _tc_final(
        bp, dinv8, self2, batch.reshape(NB, 1, RB), graph_feat, Wlin,
        blin.reshape(1, OUT),
    )


# speedup vs baseline: 1.0536x; 1.0085x over previous
import functools

import jax
import jax.numpy as jnp
from jax import lax
from jax.experimental import pallas as pl
from jax.experimental.pallas import tpu as pltpu
from jax.experimental.pallas import tpu_sc as plsc

N = 10000
E = 320000
D = 128
H = 128
G = 16
GF = 1
OUT = 1

NC = 2
NS = 16
NW = NC * NS
EPW = E // NW
CH = 80
NCH = EPW // CH
CHS = 40
PADE = 0
EPWP = EPW + PADE
NCHS = EPWP // CHS
NPR = NCHS // 2
RPT = 624
NTAIL = N - NS * RPT

_F32 = jnp.float32
_HIGH = lax.Precision.HIGHEST

@functools.cache
def _mesh():
    return plsc.VectorSubcoreMesh(
        core_axis_name="c", subcore_axis_name="s", num_cores=NC, num_subcores=NS
    )


def _sc_degree(dst, ones128, z128):
    @functools.partial(
        pl.kernel,
        out_type=jax.ShapeDtypeStruct((NC, N, D), _F32),
        mesh=_mesh(),
        scratch_types=[
            pltpu.VMEM((NCH, CH), jnp.int32),
            pltpu.VMEM((CH, D), _F32),
            pltpu.VMEM_SHARED((N, D), _F32),
            pltpu.SemaphoreType.DMA,
        ],
    )
    def deg_kernel(dst_hbm, ones_hbm, z16_hbm, out_hbm, dsts, ones_v, hist, sem):
        c = lax.axis_index("c")
        s = lax.axis_index("s")
        wid = c * NS + s
        pltpu.sync_copy(z16_hbm.at[pl.ds(s * RPT, RPT)], hist.at[pl.ds(s * RPT, RPT)])

        @pl.when(s == NS - 1)
        def _():
            pltpu.sync_copy(
                z16_hbm.at[pl.ds(NS * RPT, NTAIL)], hist.at[pl.ds(NS * RPT, NTAIL)]
            )

        pltpu.sync_copy(ones_hbm, ones_v)
        pltpu.sync_copy(dst_hbm.at[wid], dsts)
        plsc.subcore_barrier()

        @pl.loop(0, NCH)
        def _(k):
            pltpu.async_copy(ones_v, hist.at[dsts.at[k]], sem, add=True)

        @pl.loop(0, NCH)
        def _(k):
            pltpu.make_async_copy(ones_v, hist.at[dsts.at[k]], sem).wait()

        plsc.subcore_barrier()
        pltpu.sync_copy(
            hist.at[pl.ds(s * RPT, RPT)],
            out_hbm.at[c, pl.ds(s * RPT, RPT)],
        )

        @pl.when(s == NS - 1)
        def _():
            pltpu.sync_copy(
                hist.at[pl.ds(NS * RPT, NTAIL)],
                out_hbm.at[c, pl.ds(NS * RPT, NTAIL)],
            )

    return deg_kernel(dst, ones128, z128)


def _sc_scatter(hs, eidx, z128):
    @functools.partial(
        pl.kernel,
        out_type=jax.ShapeDtypeStruct((NC, N, D), _F32),
        mesh=_mesh(),
        scratch_types=[
            pltpu.VMEM((2, 2, 2, CHS), jnp.int32),
            pltpu.VMEM((2, CHS, D), _F32),
            pltpu.VMEM_SHARED((N + 8, D), _F32),
            pltpu.SemaphoreType.DMA,
            pltpu.SemaphoreType.DMA,
            pltpu.SemaphoreType.DMA,
            pltpu.SemaphoreType.DMA,
        ],
    )
    def scatter_kernel(
        hs_hbm, eidx_hbm, z_hbm, out_hbm, ei, rows, acc, sem0, sem1, semi0, semi1
    ):
        c = lax.axis_index("c")
        s = lax.axis_index("s")
        wid = c * NS + s
        pltpu.sync_copy(z_hbm.at[pl.ds(s * RPT, RPT)], acc.at[pl.ds(s * RPT, RPT)])

        @pl.when(s == NS - 1)
        def _():
            pltpu.sync_copy(
                z_hbm.at[pl.ds(NS * RPT, NTAIL)], acc.at[pl.ds(NS * RPT, NTAIL)]
            )

        plsc.subcore_barrier()

        pltpu.sync_copy(eidx_hbm.at[wid, pl.ds(0, 2)], ei.at[0])
        pltpu.async_copy(eidx_hbm.at[wid, pl.ds(2, 2)], ei.at[1], semi1)
        pltpu.async_copy(hs_hbm.at[ei.at[0, 0, 0]], rows.at[0], sem0)

        @pl.loop(0, NPR - 1, step=2)
        def _(p):
            pltpu.make_async_copy(hs_hbm.at[ei.at[0, 0, 0]], rows.at[0], sem0).wait()
            pltpu.async_copy(hs_hbm.at[ei.at[0, 1, 0]], rows.at[1], sem1)
            pltpu.sync_copy(rows.at[0], acc.at[ei.at[0, 0, 1]], add=True)
            pltpu.make_async_copy(
                eidx_hbm.at[wid, pl.ds(2 * (p + 1), 2)], ei.at[1], semi1
            ).wait()
            pltpu.make_async_copy(hs_hbm.at[ei.at[0, 1, 0]], rows.at[1], sem1).wait()
            pltpu.async_copy(hs_hbm.at[ei.at[1, 0, 0]], rows.at[0], sem0)
            pltpu.sync_copy(rows.at[1], acc.at[ei.at[0, 1, 1]], add=True)
            pltpu.async_copy(eidx_hbm.at[wid, pl.ds(2 * (p + 2), 2)], ei.at[0], semi0)
            pltpu.make_async_copy(hs_hbm.at[ei.at[1, 0, 0]], rows.at[0], sem0).wait()
            pltpu.async_copy(hs_hbm.at[ei.at[1, 1, 0]], rows.at[1], sem1)
            pltpu.sync_copy(rows.at[0], acc.at[ei.at[1, 0, 1]], add=True)
            pltpu.make_async_copy(
                eidx_hbm.at[wid, pl.ds(2 * (p + 2), 2)], ei.at[0], semi0
            ).wait()
            pltpu.make_async_copy(hs_hbm.at[ei.at[1, 1, 0]], rows.at[1], sem1).wait()
            pltpu.async_copy(hs_hbm.at[ei.at[0, 0, 0]], rows.at[0], sem0)
            pltpu.sync_copy(rows.at[1], acc.at[ei.at[1, 1, 1]], add=True)

            @pl.when(p + 3 < NPR)
            def _():
                pltpu.async_copy(
                    eidx_hbm.at[wid, pl.ds(2 * (p + 3), 2)], ei.at[1], semi1
                )

        pltpu.make_async_copy(hs_hbm.at[ei.at[0, 0, 0]], rows.at[0], sem0).wait()
        pltpu.async_copy(hs_hbm.at[ei.at[0, 1, 0]], rows.at[1], sem1)
        pltpu.sync_copy(rows.at[0], acc.at[ei.at[0, 0, 1]], add=True)
        pltpu.make_async_copy(hs_hbm.at[ei.at[0, 1, 0]], rows.at[1], sem1).wait()
        pltpu.sync_copy(rows.at[1], acc.at[ei.at[0, 1, 1]], add=True)

        plsc.subcore_barrier()
        pltpu.sync_copy(
            acc.at[pl.ds(s * RPT, RPT)],
            out_hbm.at[c, pl.ds(s * RPT, RPT)],
        )

        @pl.when(s == NS - 1)
        def _():
            pltpu.sync_copy(
                acc.at[pl.ds(NS * RPT, NTAIL)],
                out_hbm.at[c, pl.ds(NS * RPT, NTAIL)],
            )

    return scatter_kernel(hs, eidx, z128)


RB = 1000
NB = N // RB

_row = pl.BlockSpec((RB, H), lambda i: (i, 0))
_degs = pl.BlockSpec((NC, RB, D), lambda i: (0, i, 0))
_part = pl.BlockSpec((NC, RB, D), lambda i: (0, i, 0))
_wspec = pl.BlockSpec((H, H), lambda i: (0, 0))
_bspec = pl.BlockSpec((1, H), lambda i: (0, 0))


def _deg_cols(degp_ref):
    deg = degp_ref[0, :, 0:1] + degp_ref[1, :, 0:1] + 1.0
    dinv = lax.rsqrt(deg)
    ideg = 1.0 / deg
    return dinv, ideg


def _mm_body(x_ref, w_ref, o_ref):
    o_ref[...] = jnp.dot(x_ref[...], w_ref[...], preferred_element_type=_F32)


def _tc_mm(x, W1):
    return pl.pallas_call(
        _mm_body,
        grid=(NB,),
        in_specs=[pl.BlockSpec((RB, D), lambda i: (i, 0)), _wspec],
        out_specs=_row,
        out_shape=jax.ShapeDtypeStruct((N, H), _F32),
    )(x, W1)


def _prep_body(h_ref, degp_ref, b_ref, hs_ref, self_ref, dinv_ref):
    dinv, ideg = _deg_cols(degp_ref)
    h = h_ref[...]
    hs_ref[...] = h * dinv
    self_ref[...] = h * ideg + b_ref[...]
    dinv_ref[...] = jnp.broadcast_to(dinv, (RB, 8))


def _tc_prep(h1, degp, b1):
    return pl.pallas_call(
        _prep_body,
        grid=(NB,),
        in_specs=[_row, _degs, _bspec],
        out_specs=(_row, _row, pl.BlockSpec((RB, 8), lambda i: (i, 0))),
        out_shape=(
            jax.ShapeDtypeStruct((N, H), _F32),
            jax.ShapeDtypeStruct((N, H), _F32),
            jax.ShapeDtypeStruct((N, 8), _F32),
        ),
    )(h1, degp, b1)


def _combine_body(ap_ref, dinv_ref, self1_ref, w2_ref, b2_ref, hs2_ref, self2_ref):
    dinv = dinv_ref[:, 0:1]
    ideg = dinv * dinv
    a = ap_ref[0] + ap_ref[1]
    h2in = jnp.maximum(dinv * a + self1_ref[...], 0.0)
    h2 = jnp.dot(h2in, w2_ref[...], preferred_element_type=_F32)
    hs2_ref[...] = h2 * dinv
    self2_ref[...] = h2 * ideg + b2_ref[...]


def _tc_combine(ap, dinv8, self1, W2, b2):
    return pl.pallas_call(
        _combine_body,
        grid=(NB,),
        in_specs=[_part, pl.BlockSpec((RB, 8), lambda i: (i, 0)), _row, _wspec, _bspec],
        out_specs=(_row, _row),
        out_shape=(
            jax.ShapeDtypeStruct((N, H), _F32),
            jax.ShapeDtypeStruct((N, H), _F32),
        ),
    )(ap, dinv8, self1, W2, b2)


def _final_body(
    bp_ref, dinv_ref, self2_ref, batch_ref, gf_ref, wl_ref, bl_ref, o_ref, pacc
):
    i = pl.program_id(0)
    dinv = dinv_ref[:, 0:1]
    b = bp_ref[0] + bp_ref[1]
    out2 = jnp.maximum(dinv * b + self2_ref[...], 0.0)
    seg = batch_ref[0]
    gids = lax.broadcasted_iota(jnp.int32, (G, RB), 0)
    M = (seg == gids).astype(_F32)
    part = jnp.dot(M, out2, precision=_HIGH, preferred_element_type=_F32)

    @pl.when(i == 0)
    def _():
        pacc[...] = jnp.zeros((G, H), _F32)

    pacc[...] += part

    @pl.when(i == NB - 1)
    def _():
        z = jnp.concatenate([pacc[...], gf_ref[...]], axis=1)
        o_ref[...] = jnp.dot(z, wl_ref[...], preferred_element_type=_F32) + bl_ref[...]


def _tc_final(bp, dinv8, self2, batch2d, graph_feat, Wlin, blin2d):
    return pl.pallas_call(
        _final_body,
        grid=(NB,),
        in_specs=[
            _part,
            pl.BlockSpec((RB, 8), lambda i: (i, 0)),
            _row,
            pl.BlockSpec((1, 1, RB), lambda i: (i, 0, 0)),
            pl.BlockSpec((G, GF), lambda i: (0, 0)),
            pl.BlockSpec((H + GF, OUT), lambda i: (0, 0)),
            pl.BlockSpec((1, OUT), lambda i: (0, 0)),
        ],
        out_specs=pl.BlockSpec((G, OUT), lambda i: (0, 0)),
        out_shape=jax.ShapeDtypeStruct((G, OUT), _F32),
        scratch_shapes=[pltpu.VMEM((G, H), _F32)],
    )(bp, dinv8, self2, batch2d, graph_feat, Wlin, blin2d)


def kernel(x, edge_index, batch, graph_feat, W1, b1, W2, b2, Wlin, blin):
    dst = edge_index[1].reshape(NW, NCH, CH)
    ew = edge_index.reshape(2, NW, EPW).transpose(1, 2, 0)
    if PADE:
        pad_s = jnp.zeros((NW, PADE, 1, 1), jnp.int32)
        pad_d = jnp.full((NW, PADE, 1, 1), N, jnp.int32)
        pad = jnp.concatenate([pad_s, pad_d], 3).reshape(NW, PADE, 2)
        ew = jnp.concatenate([ew, pad], 1)
    eidx = ew.reshape(NW, NCHS, CHS, 2).transpose(0, 1, 3, 2)
    z128 = jnp.zeros((N, D), _F32)
    ones128 = jnp.ones((CH, D), _F32)

    degp = _sc_degree(dst, ones128, z128)
    h1 = _tc_mm(x, W1)
    hs1, self1, dinv8 = _tc_prep(h1, degp, b1.reshape(1, H))
    ap = _sc_scatter(hs1, eidx, z128)
    hs2, self2 = _tc_combine(ap, dinv8, self1, W2, b2.reshape(1, H))
    bp = _sc_scatter(hs2, eidx, z128)
    return _tc_final(
        bp, dinv8, self2, batch.reshape(NB, 1, RB), graph_feat, Wlin,
        blin.reshape(1, OUT),
    )
